# baseline (speedup 1.0000x reference)
import jax
import jax.numpy as jnp
from jax.experimental import pallas as pl
from jax.experimental.pallas import tpu as pltpu

_BB = 8
_HI = jax.lax.Precision.DEFAULT


def _enhance_block_kernel(qr_ref, par_ref, in_ref, out_ref, row_ref, sem):
    bb, rh, d2 = in_ref.shape
    d = d2 // 2
    b0 = pl.program_id(0) * bb
    threshold = par_ref[0]
    strength = par_ref[1]
    temp = par_ref[2]
    sws = par_ref[3]

    blk = in_ref[...]
    sq = blk * blk

    lane = jax.lax.broadcasted_iota(jnp.int32, (1, d2), 1)
    lanef = jnp.where(lane < d, 1.0, 0.0)

    ids = jax.lax.broadcasted_iota(jnp.int32, (1, rh), 1)
    q_rows, oh_e, oh_o, rows128, evfs = [], [], [], [], []
    for i in range(bb):
        qri = qr_ref[b0 + i]
        qh = qri // 2
        evf = jnp.float32(1.0) - (qri % 2).astype(jnp.float32)
        evfs.append(evf)
        row = in_ref[pl.ds(i, 1), pl.ds(qh, 1), :]
        row = row[0]
        rows128.append(row)
        q_rows.append(evf * row[:, :d] + (1.0 - evf) * row[:, d:])
        hit = jnp.where(ids == qh, 1.0, 0.0)
        oh_e.append(hit * evf)
        oh_o.append(hit * (1.0 - evf))
    q = jnp.concatenate(q_rows, axis=0)
    isq_e = jnp.concatenate(oh_e, axis=0)
    isq_o = jnp.concatenate(oh_o, axis=0)

    qn = q / jnp.maximum(jnp.sqrt(jnp.sum(q * q, axis=1, keepdims=True)), 1e-12)
    qcat = jnp.concatenate([qn, qn], axis=1)[:, None, :]

    rhs_q = jnp.concatenate([qcat * lanef, qcat * (1.0 - lanef)], axis=1)
    rhs_1 = jnp.broadcast_to(
        jnp.concatenate([lanef, 1.0 - lanef], axis=0)[None], (bb, 2, d2))
    dots = jax.lax.dot_general(rhs_q, blk, (((2,), (2,)), ((0,), (0,))),
                               precision=_HI)
    rsum = jax.lax.dot_general(rhs_1, sq, (((2,), (2,)), ((0,), (0,))),
                               precision=_HI)

    inv_e = 1.0 / jnp.maximum(jnp.sqrt(rsum[:, 0, :]), 1e-12)
    inv_o = 1.0 / jnp.maximum(jnp.sqrt(rsum[:, 1, :]), 1e-12)
    sims_e = jnp.where(isq_e > 0.5, -1.0, dots[:, 0, :] * inv_e)
    sims_o = jnp.where(isq_o > 0.5, -1.0, dots[:, 1, :] * inv_o)

    def half_chain(sims):
        sim_w = jax.nn.sigmoid((sims - threshold) * 10.0)
        expw = jnp.where(sim_w > 0.5, jnp.exp(sims / temp), 0.0)
        return sim_w, expw

    simw_e, expw_e = half_chain(sims_e)
    simw_o, expw_o = half_chain(sims_o)
    denom = (jnp.sum(expw_e, axis=1, keepdims=True)
             + jnp.sum(expw_o, axis=1, keepdims=True))
    inv_denom = 1.0 / jnp.where(denom > 0.0, denom, 1.0)
    adj_e = expw_e * inv_denom * simw_e * (1.0 + sws * sims_e)
    adj_o = expw_o * inv_denom * simw_o * (1.0 + sws * sims_o)
    norm = (jnp.sum(adj_e, axis=1, keepdims=True)
            + jnp.sum(adj_o, axis=1, keepdims=True) + 1e-8)
    inv_norm = 1.0 / norm
    adj_e = adj_e * inv_norm
    adj_o = adj_o * inv_norm

    adjcat = jnp.concatenate([adj_e[:, None, :], adj_o[:, None, :]], axis=1)
    wsum = jax.lax.dot_general(adjcat, blk, (((2,), (1,)), ((0,), (0,))),
                               precision=_HI)
    weighted = wsum[:, 0, :d] + wsum[:, 1, d:]

    enhanced = (1.0 - strength) * q + strength * weighted
    final_q = jnp.where(denom > 0.0, enhanced, q)

    copies = []
    for i in range(bb):
        qri = qr_ref[b0 + i]
        qh = qri // 2
        evf = evfs[i]
        fq = final_q[i:i + 1]
        fq2 = jnp.concatenate([fq, fq], axis=1)
        wm = evf * lanef + (1.0 - evf) * (1.0 - lanef)
        newrow = wm * fq2 + (1.0 - wm) * rows128[i]
        row_ref[pl.ds(i, 1), 0, :] = newrow
        cp = pltpu.make_async_copy(
            row_ref.at[pl.ds(i, 1)],
            out_ref.at[pl.ds(b0 + i, 1), pl.ds(qh, 1), :],
            sem)
        cp.start()
        copies.append(cp)
    for cp in copies:
        cp.wait()


def kernel(final_relation_representations, query_rels, similarity_threshold_raw,
           enhancement_strength_raw, similarity_weight_scale, temperature):
    reprs = final_relation_representations
    b, r, d = reprs.shape
    threshold = jax.nn.sigmoid(similarity_threshold_raw)
    strength = jax.nn.sigmoid(enhancement_strength_raw) * 0.2
    temp = jnp.clip(temperature, 0.1, 10.0)
    params = jnp.stack([threshold, strength, temp,
                        jnp.float32(similarity_weight_scale)]).astype(jnp.float32)
    qr = query_rels.astype(jnp.int32)

    packed = reprs.reshape(b, r // 2, 2 * d)
    grid = (b // _BB,)
    out = pl.pallas_call(
        _enhance_block_kernel,
        grid=grid,
        in_specs=[
            pl.BlockSpec(memory_space=pltpu.SMEM),
            pl.BlockSpec(memory_space=pltpu.SMEM),
            pl.BlockSpec((_BB, r // 2, 2 * d), lambda i: (i, 0, 0)),
        ],
        out_specs=pl.BlockSpec(memory_space=pl.ANY),
        out_shape=jax.ShapeDtypeStruct((b, r // 2, 2 * d), reprs.dtype),
        scratch_shapes=[pltpu.VMEM((_BB, 1, 2 * d), jnp.float32),
                        pltpu.SemaphoreType.DMA],
        input_output_aliases={2: 0},
        compiler_params=pltpu.CompilerParams(
            dimension_semantics=("parallel",),
        ),
    )(qr, params, packed)
    return out.reshape(b, r, d)


# speedup vs baseline: 1.1624x; 1.1624x over previous
import jax
import jax.numpy as jnp
from jax.experimental import pallas as pl
from jax.experimental.pallas import tpu as pltpu

_BB = 16
_HI = jax.lax.Precision.DEFAULT


def _enhance_block_kernel(qr_ref, par_ref, in_ref, out_ref):
    bb, rh, d2 = in_ref.shape
    d = d2 // 2
    b0 = pl.program_id(0) * bb
    threshold = par_ref[0]
    strength = par_ref[1]
    temp = par_ref[2]
    sws = par_ref[3]

    blk = in_ref[...]
    sq = blk * blk

    lane = jax.lax.broadcasted_iota(jnp.int32, (1, d2), 1)
    lanef = jnp.where(lane < d, 1.0, 0.0)

    ids = jax.lax.broadcasted_iota(jnp.int32, (1, rh), 1)
    q_rows, oh_e, oh_o, rows128, evfs = [], [], [], [], []
    for i in range(bb):
        qri = qr_ref[b0 + i]
        qh = qri // 2
        evf = jnp.float32(1.0) - (qri % 2).astype(jnp.float32)
        evfs.append(evf)
        row = in_ref[pl.ds(i, 1), pl.ds(qh, 1), :]
        row = row[0]
        rows128.append(row)
        q_rows.append(evf * row[:, :d] + (1.0 - evf) * row[:, d:])
        hit = jnp.where(ids == qh, 1.0, 0.0)
        oh_e.append(hit * evf)
        oh_o.append(hit * (1.0 - evf))
    q = jnp.concatenate(q_rows, axis=0)
    isq_e = jnp.concatenate(oh_e, axis=0)
    isq_o = jnp.concatenate(oh_o, axis=0)

    qn = q / jnp.maximum(jnp.sqrt(jnp.sum(q * q, axis=1, keepdims=True)), 1e-12)
    qcat = jnp.concatenate([qn, qn], axis=1)[:, None, :]

    rhs_q = jnp.concatenate([qcat * lanef, qcat * (1.0 - lanef)], axis=1)
    rhs_1 = jnp.broadcast_to(
        jnp.concatenate([lanef, 1.0 - lanef], axis=0)[None], (bb, 2, d2))
    dots = jax.lax.dot_general(rhs_q, blk, (((2,), (2,)), ((0,), (0,))),
                               precision=_HI)
    rsum = jax.lax.dot_general(rhs_1, sq, (((2,), (2,)), ((0,), (0,))),
                               precision=_HI)

    inv_e = 1.0 / jnp.maximum(jnp.sqrt(rsum[:, 0, :]), 1e-12)
    inv_o = 1.0 / jnp.maximum(jnp.sqrt(rsum[:, 1, :]), 1e-12)
    sims_e = jnp.where(isq_e > 0.5, -1.0, dots[:, 0, :] * inv_e)
    sims_o = jnp.where(isq_o > 0.5, -1.0, dots[:, 1, :] * inv_o)

    def half_chain(sims):
        sim_w = jax.nn.sigmoid((sims - threshold) * 10.0)
        expw = jnp.where(sim_w > 0.5, jnp.exp(sims / temp), 0.0)
        return sim_w, expw

    simw_e, expw_e = half_chain(sims_e)
    simw_o, expw_o = half_chain(sims_o)
    denom = (jnp.sum(expw_e, axis=1, keepdims=True)
             + jnp.sum(expw_o, axis=1, keepdims=True))
    inv_denom = 1.0 / jnp.where(denom > 0.0, denom, 1.0)
    adj_e = expw_e * inv_denom * simw_e * (1.0 + sws * sims_e)
    adj_o = expw_o * inv_denom * simw_o * (1.0 + sws * sims_o)
    norm = (jnp.sum(adj_e, axis=1, keepdims=True)
            + jnp.sum(adj_o, axis=1, keepdims=True) + 1e-8)
    inv_norm = 1.0 / norm
    adj_e = adj_e * inv_norm
    adj_o = adj_o * inv_norm

    adjcat = jnp.concatenate([adj_e[:, None, :], adj_o[:, None, :]], axis=1)
    wsum = jax.lax.dot_general(adjcat, blk, (((2,), (1,)), ((0,), (0,))),
                               precision=_HI)
    weighted = wsum[:, 0, :d] + wsum[:, 1, d:]

    enhanced = (1.0 - strength) * q + strength * weighted
    final_q = jnp.where(denom > 0.0, enhanced, q)

    out_ref[...] = blk
    for i in range(bb):
        qri = qr_ref[b0 + i]
        qh = qri // 2
        evf = evfs[i]
        fq = final_q[i:i + 1]
        fq2 = jnp.concatenate([fq, fq], axis=1)
        wm = evf * lanef + (1.0 - evf) * (1.0 - lanef)
        newrow = wm * fq2 + (1.0 - wm) * rows128[i]
        out_ref[pl.ds(i, 1), pl.ds(qh, 1), :] = newrow[None]


def kernel(final_relation_representations, query_rels, similarity_threshold_raw,
           enhancement_strength_raw, similarity_weight_scale, temperature):
    reprs = final_relation_representations
    b, r, d = reprs.shape
    threshold = jax.nn.sigmoid(similarity_threshold_raw)
    strength = jax.nn.sigmoid(enhancement_strength_raw) * 0.2
    temp = jnp.clip(temperature, 0.1, 10.0)
    params = jnp.stack([threshold, strength, temp,
                        jnp.float32(similarity_weight_scale)]).astype(jnp.float32)
    qr = query_rels.astype(jnp.int32)

    packed = reprs.reshape(b, r // 2, 2 * d)
    bbs = _BB if b % _BB == 0 else 1
    grid = (b // bbs,)
    out = pl.pallas_call(
        _enhance_block_kernel,
        grid=grid,
        in_specs=[
            pl.BlockSpec(memory_space=pltpu.SMEM),
            pl.BlockSpec(memory_space=pltpu.SMEM),
            pl.BlockSpec((bbs, r // 2, 2 * d), lambda i: (i, 0, 0)),
        ],
        out_specs=pl.BlockSpec((bbs, r // 2, 2 * d), lambda i: (i, 0, 0)),
        out_shape=jax.ShapeDtypeStruct((b, r // 2, 2 * d), reprs.dtype),
        compiler_params=pltpu.CompilerParams(
            dimension_semantics=("parallel",),
        ),
    )(qr, params, packed)
    return out.reshape(b, r, d)


# speedup vs baseline: 1.2164x; 1.0465x over previous
import jax
import jax.numpy as jnp
from jax.experimental import pallas as pl
from jax.experimental.pallas import tpu as pltpu

_BB = 32
_HI = jax.lax.Precision.DEFAULT


def _enhance_block_kernel(qr_ref, par_ref, in_ref, out_ref):
    bb, rh, d2 = in_ref.shape
    d = d2 // 2
    b0 = pl.program_id(0) * bb
    threshold = par_ref[0]
    strength = par_ref[1]
    temp = par_ref[2]
    sws = par_ref[3]

    blk = in_ref[...]
    sq = blk * blk

    lane = jax.lax.broadcasted_iota(jnp.int32, (1, d2), 1)
    lanef = jnp.where(lane < d, 1.0, 0.0)

    ids = jax.lax.broadcasted_iota(jnp.int32, (1, rh), 1)
    q_rows, oh_e, oh_o, rows128, evfs = [], [], [], [], []
    for i in range(bb):
        qri = qr_ref[b0 + i]
        qh = qri // 2
        evf = jnp.float32(1.0) - (qri % 2).astype(jnp.float32)
        evfs.append(evf)
        row = in_ref[pl.ds(i, 1), pl.ds(qh, 1), :]
        row = row[0]
        rows128.append(row)
        q_rows.append(evf * row[:, :d] + (1.0 - evf) * row[:, d:])
        hit = jnp.where(ids == qh, 1.0, 0.0)
        oh_e.append(hit * evf)
        oh_o.append(hit * (1.0 - evf))
    q = jnp.concatenate(q_rows, axis=0)
    isq_e = jnp.concatenate(oh_e, axis=0)
    isq_o = jnp.concatenate(oh_o, axis=0)

    qn = q / jnp.maximum(jnp.sqrt(jnp.sum(q * q, axis=1, keepdims=True)), 1e-12)
    qcat = jnp.concatenate([qn, qn], axis=1)[:, None, :]

    rhs_q = jnp.concatenate([qcat * lanef, qcat * (1.0 - lanef)], axis=1)
    rhs_1 = jnp.broadcast_to(
        jnp.concatenate([lanef, 1.0 - lanef], axis=0)[None], (bb, 2, d2))
    dots = jax.lax.dot_general(rhs_q, blk, (((2,), (2,)), ((0,), (0,))),
                               precision=_HI)
    rsum = jax.lax.dot_general(rhs_1, sq, (((2,), (2,)), ((0,), (0,))),
                               precision=_HI)

    inv_e = 1.0 / jnp.maximum(jnp.sqrt(rsum[:, 0, :]), 1e-12)
    inv_o = 1.0 / jnp.maximum(jnp.sqrt(rsum[:, 1, :]), 1e-12)
    sims_e = jnp.where(isq_e > 0.5, -1.0, dots[:, 0, :] * inv_e)
    sims_o = jnp.where(isq_o > 0.5, -1.0, dots[:, 1, :] * inv_o)

    def half_chain(sims):
        sim_w = jax.nn.sigmoid((sims - threshold) * 10.0)
        expw = jnp.where(sim_w > 0.5, jnp.exp(sims / temp), 0.0)
        return sim_w, expw

    simw_e, expw_e = half_chain(sims_e)
    simw_o, expw_o = half_chain(sims_o)
    denom = (jnp.sum(expw_e, axis=1, keepdims=True)
             + jnp.sum(expw_o, axis=1, keepdims=True))
    inv_denom = 1.0 / jnp.where(denom > 0.0, denom, 1.0)
    adj_e = expw_e * inv_denom * simw_e * (1.0 + sws * sims_e)
    adj_o = expw_o * inv_denom * simw_o * (1.0 + sws * sims_o)
    norm = (jnp.sum(adj_e, axis=1, keepdims=True)
            + jnp.sum(adj_o, axis=1, keepdims=True) + 1e-8)
    inv_norm = 1.0 / norm
    adj_e = adj_e * inv_norm
    adj_o = adj_o * inv_norm

    adjcat = jnp.concatenate([adj_e[:, None, :], adj_o[:, None, :]], axis=1)
    wsum = jax.lax.dot_general(adjcat, blk, (((2,), (1,)), ((0,), (0,))),
                               precision=_HI)
    weighted = wsum[:, 0, :d] + wsum[:, 1, d:]

    enhanced = (1.0 - strength) * q + strength * weighted
    final_q = jnp.where(denom > 0.0, enhanced, q)

    out_ref[...] = blk
    for i in range(bb):
        qri = qr_ref[b0 + i]
        qh = qri // 2
        evf = evfs[i]
        fq = final_q[i:i + 1]
        fq2 = jnp.concatenate([fq, fq], axis=1)
        wm = evf * lanef + (1.0 - evf) * (1.0 - lanef)
        newrow = wm * fq2 + (1.0 - wm) * rows128[i]
        out_ref[pl.ds(i, 1), pl.ds(qh, 1), :] = newrow[None]


def kernel(final_relation_representations, query_rels, similarity_threshold_raw,
           enhancement_strength_raw, similarity_weight_scale, temperature):
    reprs = final_relation_representations
    b, r, d = reprs.shape
    threshold = jax.nn.sigmoid(similarity_threshold_raw)
    strength = jax.nn.sigmoid(enhancement_strength_raw) * 0.2
    temp = jnp.clip(temperature, 0.1, 10.0)
    params = jnp.stack([threshold, strength, temp,
                        jnp.float32(similarity_weight_scale)]).astype(jnp.float32)
    qr = query_rels.astype(jnp.int32)

    packed = reprs.reshape(b, r // 2, 2 * d)
    bbs = _BB if b % _BB == 0 else 1
    grid = (b // bbs,)
    out = pl.pallas_call(
        _enhance_block_kernel,
        grid=grid,
        in_specs=[
            pl.BlockSpec(memory_space=pltpu.SMEM),
            pl.BlockSpec(memory_space=pltpu.SMEM),
            pl.BlockSpec((bbs, r // 2, 2 * d), lambda i: (i, 0, 0)),
        ],
        out_specs=pl.BlockSpec((bbs, r // 2, 2 * d), lambda i: (i, 0, 0)),
        out_shape=jax.ShapeDtypeStruct((b, r // 2, 2 * d), reprs.dtype),
        compiler_params=pltpu.CompilerParams(
            dimension_semantics=("parallel",),
        ),
    )(qr, params, packed)
    return out.reshape(b, r, d)
